# baseline (speedup 1.0000x reference)
import functools

import jax
import jax.numpy as jnp
from jax import lax
from jax.experimental import pallas as pl
from jax.experimental.pallas import tpu as pltpu
from jax.experimental.pallas import tpu_sc as plsc

N = 10816
E = 346112
D = 128
G = 64
NPG = 169
NC = 2
NS = 16
NW = NC * NS
EPW = E // NW
CHUNK = 128
NCH = 86
NPAD = NCH * CHUNK - EPW
NACC = N + 64
RBLK = 64
NBLK = N // RBLK

_HI = lax.Precision.HIGHEST


def _sc_aggregate(h, epk):
    mesh = plsc.VectorSubcoreMesh(core_axis_name="c", subcore_axis_name="s")

    @functools.partial(
        pl.kernel,
        out_type=jax.ShapeDtypeStruct((NC, N, D), jnp.float32),
        mesh=mesh,
        scratch_types=[
            pltpu.VMEM((NCH, CHUNK), jnp.int32),
            pltpu.VMEM((2, CHUNK), jnp.int32),
            pltpu.VMEM((2, CHUNK), jnp.int32),
            pltpu.VMEM((1, CHUNK, D), jnp.float32),
            pltpu.VMEM_SHARED((NACC, D), jnp.float32),
            pltpu.SemaphoreType.DMA,
            pltpu.SemaphoreType.DMA,
            pltpu.SemaphoreType.DMA,
            pltpu.SemaphoreType.DMA,
        ],
    )
    def k(h_hbm, epk_hbm, out_hbm, epk_v, src_v, dst_v, rows_v, agg_sh,
          gsem0, gsem1, ssem0, ssem1):
        c = lax.axis_index("c")
        s = lax.axis_index("s")
        wid = c * NS + s
        gsem = (gsem0, gsem1)
        ssem = (ssem0, ssem1)

        pltpu.sync_copy(epk_hbm.at[wid], epk_v)

        mask16 = jnp.full((16,), 0xFFFF, jnp.int32)

        def unpack(j, b):
            for cc in range(0, CHUNK, 16):
                v = epk_v[j, pl.ds(cc, 16)]
                src_v[b, pl.ds(cc, 16)] = jnp.bitwise_and(v, mask16)
                dst_v[b, pl.ds(cc, 16)] = lax.shift_right_logical(v, 16)

        def gather(b):
            return pltpu.make_async_copy(
                h_hbm.at[src_v.at[b]], rows_v.at[b], gsem[b])

        def scatter(b):
            return pltpu.make_async_copy(
                rows_v.at[b], agg_sh.at[dst_v.at[b]], ssem[b])

        @pl.loop(0, CHUNK)
        def _(r):
            for cc in range(0, D, 16):
                rows_v[0, r, pl.ds(cc, 16)] = jnp.zeros((16,), jnp.float32)

        @pl.loop(0, 7)
        def _(t):
            b = s + NS * t

            @pl.when(b < NACC // CHUNK)
            def _():
                pltpu.sync_copy(rows_v.at[0], agg_sh.at[pl.ds(b * CHUNK,
                                                              CHUNK)])

        plsc.subcore_barrier()

        @pl.loop(0, NCH)
        def _(j):
            unpack(j, 0)
            gather(0).start()
            gather(0).wait()
            scatter(0).start(add=True)
            scatter(0).wait()

        plsc.subcore_barrier()

        @pl.loop(0, 11)
        def _(t):
            b = s + NS * t

            @pl.when(b < NBLK)
            def _():
                pltpu.sync_copy(agg_sh.at[pl.ds(b * RBLK, RBLK)],
                                out_hbm.at[c].at[pl.ds(b * RBLK, RBLK)])

    return k(h, epk)


MLP_BLK = 1352


def _mlp(p, h, W1, b1, W2, b2):
    def body(p_ref, h_ref, w1_ref, b1_ref, w2_ref, b2_ref, o_ref):
        z = p_ref[0] + p_ref[1] + h_ref[...]
        z = jnp.dot(z, w1_ref[...], precision=_HI,
                    preferred_element_type=jnp.float32) + b1_ref[...]
        z = jnp.maximum(z, 0.0)
        z = jnp.dot(z, w2_ref[...], precision=_HI,
                    preferred_element_type=jnp.float32) + b2_ref[...]
        o_ref[...] = jnp.maximum(z, 0.0)

    return pl.pallas_call(
        body,
        grid=(N // MLP_BLK,),
        in_specs=[
            pl.BlockSpec((NC, MLP_BLK, D), lambda i: (0, i, 0)),
            pl.BlockSpec((MLP_BLK, D), lambda i: (i, 0)),
            pl.BlockSpec((D, D), lambda i: (0, 0)),
            pl.BlockSpec((1, D), lambda i: (0, 0)),
            pl.BlockSpec((D, D), lambda i: (0, 0)),
            pl.BlockSpec((1, D), lambda i: (0, 0)),
        ],
        out_specs=pl.BlockSpec((MLP_BLK, D), lambda i: (i, 0)),
        out_shape=jax.ShapeDtypeStruct((N, D), jnp.float32),
    )(p, h, W1, b1, W2, b2)


def _head(h, Wl, bl, Wm, bm, Ws, bs):
    def body(h_ref, wl_ref, bl_ref, wm_ref, bm_ref, ws_ref, bs_ref,
             mean_ref, std_ref):
        node = lax.broadcasted_iota(jnp.int32, (G, N), 1)
        gid = lax.broadcasted_iota(jnp.int32, (G, N), 0)
        P = jnp.where(node // NPG == gid, 1.0, 0.0)
        pooled = jnp.dot(P, h_ref[...], precision=_HI,
                         preferred_element_type=jnp.float32) * (1.0 / NPG)
        feat = jnp.dot(pooled, wl_ref[...], precision=_HI,
                       preferred_element_type=jnp.float32) + bl_ref[...]
        feat = jnp.maximum(feat, 0.0)
        mean_ref[...] = jnp.dot(feat, wm_ref[...], precision=_HI,
                                preferred_element_type=jnp.float32) + bm_ref[...]
        sv = jnp.dot(feat, ws_ref[...], precision=_HI,
                     preferred_element_type=jnp.float32) + bs_ref[...]
        std_ref[...] = jax.nn.softplus(sv)

    return pl.pallas_call(
        body,
        out_shape=(jax.ShapeDtypeStruct((G, 32), jnp.float32),
                   jax.ShapeDtypeStruct((G, 32), jnp.float32)),
    )(h, Wl, bl, Wm, bm, Ws, bs)


def kernel(x, edge_index, W1, b1, W2, b2, Wl, bl, Wm, bm, Ws, bs):
    src = edge_index[0].reshape(NW, EPW)
    dst = edge_index[1].reshape(NW, EPW)
    packed = jnp.bitwise_or(src, jnp.left_shift(dst, 16))
    pad = jnp.broadcast_to(
        jnp.left_shift(N + jnp.arange(NPAD, dtype=jnp.int32) % 64,
                       16)[None, :],
        (NW, NPAD))
    epk = jnp.concatenate([packed, pad], axis=1).reshape(NW, NCH, CHUNK)
    h = x
    for l in range(3):
        p = _sc_aggregate(h, epk)
        h = _mlp(p, h, W1[l], b1[l].reshape(1, D), W2[l], b2[l].reshape(1, D))
    mean, std = _head(h, Wl, bl.reshape(1, -1), Wm, bm.reshape(1, -1),
                      Ws, bs.reshape(1, -1))
    return (mean, std)


# speedup vs baseline: 1.0024x; 1.0024x over previous
import functools

import jax
import jax.numpy as jnp
from jax import lax
from jax.experimental import pallas as pl
from jax.experimental.pallas import tpu as pltpu
from jax.experimental.pallas import tpu_sc as plsc

N = 10816
E = 346112
D = 128
G = 64
NPG = 169
NC = 2
NS = 16
NW = NC * NS
EPW = E // NW
CHUNK = 128
NCH = 86
NPAD = NCH * CHUNK - EPW
NACC = N + 64
RBLK = 64
NBLK = N // RBLK

_HI = lax.Precision.HIGHEST


def _sc_aggregate(h, epk):
    mesh = plsc.VectorSubcoreMesh(core_axis_name="c", subcore_axis_name="s")

    @functools.partial(
        pl.kernel,
        out_type=jax.ShapeDtypeStruct((NC, N, D), jnp.float32),
        mesh=mesh,
        scratch_types=[
            pltpu.VMEM((NCH, CHUNK), jnp.int32),
            pltpu.VMEM((1, CHUNK), jnp.int32),
            pltpu.VMEM((1, CHUNK), jnp.int32),
            pltpu.VMEM((CHUNK, D), jnp.float32),
            pltpu.VMEM_SHARED((NACC, D), jnp.float32),
        ],
    )
    def k(h_hbm, epk_hbm, out_hbm, epk_v, src_v, dst_v, rows_v, agg_sh):
        c = lax.axis_index("c")
        s = lax.axis_index("s")
        wid = c * NS + s

        pltpu.sync_copy(epk_hbm.at[wid], epk_v)

        mask16 = jnp.full((16,), 0xFFFF, jnp.int32)

        @pl.loop(0, CHUNK)
        def _(r):
            for cc in range(0, D, 16):
                rows_v[r, pl.ds(cc, 16)] = jnp.zeros((16,), jnp.float32)

        @pl.loop(0, 7)
        def _(t):
            b = s + NS * t

            @pl.when(b < NACC // CHUNK)
            def _():
                pltpu.sync_copy(rows_v, agg_sh.at[pl.ds(b * CHUNK, CHUNK)])

        plsc.subcore_barrier()

        @pl.loop(0, NCH)
        def _(j):
            for cc in range(0, CHUNK, 16):
                v = epk_v[j, pl.ds(cc, 16)]
                src_v[0, pl.ds(cc, 16)] = jnp.bitwise_and(v, mask16)
                dst_v[0, pl.ds(cc, 16)] = lax.shift_right_logical(v, 16)
            pltpu.sync_copy(h_hbm.at[src_v.at[0]], rows_v)
            pltpu.sync_copy(rows_v, agg_sh.at[dst_v.at[0]], add=True)

        plsc.subcore_barrier()

        @pl.loop(0, 11)
        def _(t):
            b = s + NS * t

            @pl.when(b < NBLK)
            def _():
                pltpu.sync_copy(agg_sh.at[pl.ds(b * RBLK, RBLK)],
                                out_hbm.at[c].at[pl.ds(b * RBLK, RBLK)])

    return k(h, epk)


MLP_BLK = 1352


def _mlp(p, h, W1, b1, W2, b2):
    def body(p_ref, h_ref, w1_ref, b1_ref, w2_ref, b2_ref, o_ref):
        z = p_ref[0] + p_ref[1] + h_ref[...]
        z = jnp.dot(z, w1_ref[...], precision=_HI,
                    preferred_element_type=jnp.float32) + b1_ref[...]
        z = jnp.maximum(z, 0.0)
        z = jnp.dot(z, w2_ref[...], precision=_HI,
                    preferred_element_type=jnp.float32) + b2_ref[...]
        o_ref[...] = jnp.maximum(z, 0.0)

    return pl.pallas_call(
        body,
        grid=(N // MLP_BLK,),
        in_specs=[
            pl.BlockSpec((NC, MLP_BLK, D), lambda i: (0, i, 0)),
            pl.BlockSpec((MLP_BLK, D), lambda i: (i, 0)),
            pl.BlockSpec((D, D), lambda i: (0, 0)),
            pl.BlockSpec((1, D), lambda i: (0, 0)),
            pl.BlockSpec((D, D), lambda i: (0, 0)),
            pl.BlockSpec((1, D), lambda i: (0, 0)),
        ],
        out_specs=pl.BlockSpec((MLP_BLK, D), lambda i: (i, 0)),
        out_shape=jax.ShapeDtypeStruct((N, D), jnp.float32),
    )(p, h, W1, b1, W2, b2)


def _head(h, Wl, bl, Wm, bm, Ws, bs):
    def body(h_ref, wl_ref, bl_ref, wm_ref, bm_ref, ws_ref, bs_ref,
             mean_ref, std_ref):
        node = lax.broadcasted_iota(jnp.int32, (G, N), 1)
        gid = lax.broadcasted_iota(jnp.int32, (G, N), 0)
        P = jnp.where(node // NPG == gid, 1.0, 0.0)
        pooled = jnp.dot(P, h_ref[...], precision=_HI,
                         preferred_element_type=jnp.float32) * (1.0 / NPG)
        feat = jnp.dot(pooled, wl_ref[...], precision=_HI,
                       preferred_element_type=jnp.float32) + bl_ref[...]
        feat = jnp.maximum(feat, 0.0)
        mean_ref[...] = jnp.dot(feat, wm_ref[...], precision=_HI,
                                preferred_element_type=jnp.float32) + bm_ref[...]
        sv = jnp.dot(feat, ws_ref[...], precision=_HI,
                     preferred_element_type=jnp.float32) + bs_ref[...]
        std_ref[...] = jax.nn.softplus(sv)

    return pl.pallas_call(
        body,
        out_shape=(jax.ShapeDtypeStruct((G, 32), jnp.float32),
                   jax.ShapeDtypeStruct((G, 32), jnp.float32)),
    )(h, Wl, bl, Wm, bm, Ws, bs)


def kernel(x, edge_index, W1, b1, W2, b2, Wl, bl, Wm, bm, Ws, bs):
    src = edge_index[0].reshape(NW, EPW)
    dst = edge_index[1].reshape(NW, EPW)
    packed = jnp.bitwise_or(src, jnp.left_shift(dst, 16))
    pad = jnp.broadcast_to(
        jnp.left_shift(N + jnp.arange(NPAD, dtype=jnp.int32) % 64,
                       16)[None, :],
        (NW, NPAD))
    epk = jnp.concatenate([packed, pad], axis=1).reshape(NW, NCH, CHUNK)
    h = x
    for l in range(3):
        p = _sc_aggregate(h, epk)
        h = _mlp(p, h, W1[l], b1[l].reshape(1, D), W2[l], b2[l].reshape(1, D))
    mean, std = _head(h, Wl, bl.reshape(1, -1), Wm, bm.reshape(1, -1),
                      Ws, bs.reshape(1, -1))
    return (mean, std)


# speedup vs baseline: 1.9447x; 1.9400x over previous
import functools

import jax
import jax.numpy as jnp
from jax import lax
from jax.experimental import pallas as pl
from jax.experimental.pallas import tpu as pltpu
from jax.experimental.pallas import tpu_sc as plsc

N = 10816
E = 346112
D = 128
G = 64
NPG = 169
NC = 2
NS = 16
NW = NC * NS
EPW = E // NW
CHUNK = 128
NFULL = EPW // CHUNK
TAIL = EPW - NFULL * CHUNK
NCH = NFULL + 1
NACC = N
NPAD = NCH * CHUNK - EPW
RBLK = 64
NBLK = N // RBLK

_HI = lax.Precision.HIGHEST


def _sc_aggregate(h, epk):
    mesh = plsc.VectorSubcoreMesh(core_axis_name="c", subcore_axis_name="s")

    @functools.partial(
        pl.kernel,
        out_type=jax.ShapeDtypeStruct((NC, N, D), jnp.float32),
        mesh=mesh,
        scratch_types=[
            pltpu.VMEM((NCH, CHUNK), jnp.int32),
            pltpu.VMEM((1, CHUNK), jnp.int32),
            pltpu.VMEM((1, CHUNK), jnp.int32),
            pltpu.VMEM((CHUNK, D), jnp.float32),
            pltpu.VMEM_SHARED((NACC, D), jnp.float32),
        ],
    )
    def k(h_hbm, epk_hbm, out_hbm, epk_v, src_v, dst_v, rows_v, agg_sh):
        c = lax.axis_index("c")
        s = lax.axis_index("s")
        wid = c * NS + s

        pltpu.sync_copy(epk_hbm.at[wid], epk_v)

        mask16 = jnp.full((16,), 0xFFFF, jnp.int32)

        @pl.loop(0, CHUNK)
        def _(r):
            for cc in range(0, D, 16):
                rows_v[r, pl.ds(cc, 16)] = jnp.zeros((16,), jnp.float32)

        @pl.loop(0, 11)
        def _(t):
            b = s + NS * t

            @pl.when(b < NBLK)
            def _():
                pltpu.sync_copy(rows_v.at[pl.ds(0, RBLK)],
                                agg_sh.at[pl.ds(b * RBLK, RBLK)])

        plsc.subcore_barrier()

        def unpack(j, width):
            for cc in range(0, width, 16):
                v = epk_v[j, pl.ds(cc, 16)]
                src_v[0, pl.ds(cc, 16)] = jnp.bitwise_and(v, mask16)
                dst_v[0, pl.ds(cc, 16)] = lax.shift_right_logical(v, 16)

        @pl.loop(0, NFULL)
        def _(j):
            unpack(j, CHUNK)
            pltpu.sync_copy(h_hbm.at[src_v.at[0]], rows_v)
            pltpu.sync_copy(rows_v, agg_sh.at[dst_v.at[0]], add=True)

        unpack(NFULL, TAIL)
        pltpu.sync_copy(h_hbm.at[src_v.at[0].at[pl.ds(0, TAIL)]],
                        rows_v.at[pl.ds(0, TAIL)])
        pltpu.sync_copy(rows_v.at[pl.ds(0, TAIL)],
                        agg_sh.at[dst_v.at[0].at[pl.ds(0, TAIL)]], add=True)

        plsc.subcore_barrier()

        @pl.loop(0, 11)
        def _(t):
            b = s + NS * t

            @pl.when(b < NBLK)
            def _():
                pltpu.sync_copy(agg_sh.at[pl.ds(b * RBLK, RBLK)],
                                out_hbm.at[c].at[pl.ds(b * RBLK, RBLK)])

    return k(h, epk)


MLP_BLK = 1352


def _mlp(p, h, W1, b1, W2, b2):
    def body(p_ref, h_ref, w1_ref, b1_ref, w2_ref, b2_ref, o_ref):
        z = p_ref[0] + p_ref[1] + h_ref[...]
        z = jnp.dot(z, w1_ref[...], precision=_HI,
                    preferred_element_type=jnp.float32) + b1_ref[...]
        z = jnp.maximum(z, 0.0)
        z = jnp.dot(z, w2_ref[...], precision=_HI,
                    preferred_element_type=jnp.float32) + b2_ref[...]
        o_ref[...] = jnp.maximum(z, 0.0)

    return pl.pallas_call(
        body,
        grid=(N // MLP_BLK,),
        in_specs=[
            pl.BlockSpec((NC, MLP_BLK, D), lambda i: (0, i, 0)),
            pl.BlockSpec((MLP_BLK, D), lambda i: (i, 0)),
            pl.BlockSpec((D, D), lambda i: (0, 0)),
            pl.BlockSpec((1, D), lambda i: (0, 0)),
            pl.BlockSpec((D, D), lambda i: (0, 0)),
            pl.BlockSpec((1, D), lambda i: (0, 0)),
        ],
        out_specs=pl.BlockSpec((MLP_BLK, D), lambda i: (i, 0)),
        out_shape=jax.ShapeDtypeStruct((N, D), jnp.float32),
    )(p, h, W1, b1, W2, b2)


def _head(h, Wl, bl, Wm, bm, Ws, bs):
    def body(h_ref, wl_ref, bl_ref, wm_ref, bm_ref, ws_ref, bs_ref,
             mean_ref, std_ref):
        node = lax.broadcasted_iota(jnp.int32, (G, N), 1)
        gid = lax.broadcasted_iota(jnp.int32, (G, N), 0)
        P = jnp.where(node // NPG == gid, 1.0, 0.0)
        pooled = jnp.dot(P, h_ref[...], precision=_HI,
                         preferred_element_type=jnp.float32) * (1.0 / NPG)
        feat = jnp.dot(pooled, wl_ref[...], precision=_HI,
                       preferred_element_type=jnp.float32) + bl_ref[...]
        feat = jnp.maximum(feat, 0.0)
        mean_ref[...] = jnp.dot(feat, wm_ref[...], precision=_HI,
                                preferred_element_type=jnp.float32) + bm_ref[...]
        sv = jnp.dot(feat, ws_ref[...], precision=_HI,
                     preferred_element_type=jnp.float32) + bs_ref[...]
        std_ref[...] = jax.nn.softplus(sv)

    return pl.pallas_call(
        body,
        out_shape=(jax.ShapeDtypeStruct((G, 32), jnp.float32),
                   jax.ShapeDtypeStruct((G, 32), jnp.float32)),
    )(h, Wl, bl, Wm, bm, Ws, bs)


def kernel(x, edge_index, W1, b1, W2, b2, Wl, bl, Wm, bm, Ws, bs):
    src = edge_index[0].reshape(NW, EPW)
    dst = edge_index[1].reshape(NW, EPW)
    packed = jnp.bitwise_or(src, jnp.left_shift(dst, 16))
    pad = jnp.zeros((NW, NPAD), jnp.int32)
    epk = jnp.concatenate([packed, pad], axis=1).reshape(NW, NCH, CHUNK)
    h = x
    for l in range(3):
        p = _sc_aggregate(h, epk)
        h = _mlp(p, h, W1[l], b1[l].reshape(1, D), W2[l], b2[l].reshape(1, D))
    mean, std = _head(h, Wl, bl.reshape(1, -1), Wm, bm.reshape(1, -1),
                      Ws, bs.reshape(1, -1))
    return (mean, std)


# speedup vs baseline: 2.4550x; 1.2624x over previous
import functools

import jax
import jax.numpy as jnp
from jax import lax
from jax.experimental import pallas as pl
from jax.experimental.pallas import tpu as pltpu
from jax.experimental.pallas import tpu_sc as plsc

N = 10816
E = 346112
D = 128
G = 64
NPG = 169
NC = 2
NS = 16
NW = NC * NS
EPW = E // NW
CHUNK = 128
NFULL = EPW // CHUNK
TAIL = EPW - NFULL * CHUNK
NCH = NFULL + 1
NACC = N
NPAD = NCH * CHUNK - EPW
RBLK = 64
NBLK = N // RBLK

_HI = lax.Precision.HIGHEST


def _sc_aggregate(h, epk):
    mesh = plsc.VectorSubcoreMesh(core_axis_name="c", subcore_axis_name="s")

    @functools.partial(
        pl.kernel,
        out_type=jax.ShapeDtypeStruct((NC, N, D), jnp.float32),
        mesh=mesh,
        scratch_types=[
            pltpu.VMEM((NCH, CHUNK), jnp.int32),
            pltpu.VMEM((2, CHUNK), jnp.int32),
            pltpu.VMEM((2, CHUNK), jnp.int32),
            pltpu.VMEM((2, CHUNK, D), jnp.float32),
            pltpu.VMEM_SHARED((NACC, D), jnp.float32),
            pltpu.SemaphoreType.DMA,
            pltpu.SemaphoreType.DMA,
            pltpu.SemaphoreType.DMA,
            pltpu.SemaphoreType.DMA,
        ],
    )
    def k(h_hbm, epk_hbm, out_hbm, epk_v, src_v, dst_v, rows_v, agg_sh,
          gsem0, gsem1, ssem0, ssem1):
        c = lax.axis_index("c")
        s = lax.axis_index("s")
        wid = c * NS + s
        gsem = (gsem0, gsem1)
        ssem = (ssem0, ssem1)

        pltpu.sync_copy(epk_hbm.at[wid], epk_v)

        mask16 = jnp.full((16,), 0xFFFF, jnp.int32)

        def gather(b):
            return pltpu.make_async_copy(
                h_hbm.at[src_v.at[b]], rows_v.at[b], gsem[b])

        def scatter(b):
            return pltpu.make_async_copy(
                rows_v.at[b], agg_sh.at[dst_v.at[b]], ssem[b])

        @pl.loop(0, CHUNK)
        def _(r):
            for cc in range(0, D, 16):
                rows_v[0, r, pl.ds(cc, 16)] = jnp.zeros((16,), jnp.float32)

        @pl.loop(0, 11)
        def _(t):
            b = s + NS * t

            @pl.when(b < NBLK)
            def _():
                pltpu.sync_copy(rows_v.at[0].at[pl.ds(0, RBLK)],
                                agg_sh.at[pl.ds(b * RBLK, RBLK)])

        plsc.subcore_barrier()

        def unpack(j, b, width):
            for cc in range(0, width, 16):
                v = epk_v[j, pl.ds(cc, 16)]
                src_v[b, pl.ds(cc, 16)] = jnp.bitwise_and(v, mask16)
                dst_v[b, pl.ds(cc, 16)] = lax.shift_right_logical(v, 16)

        unpack(0, 0, CHUNK)
        gather(0).start()

        @pl.loop(0, NFULL // 2 - 1)
        def _(t):
            j0 = 2 * t
            gather(0).wait()
            unpack(j0 + 1, 1, CHUNK)
            scatter(0).start(add=True)
            gather(1).start()
            gather(1).wait()
            scatter(0).wait()
            unpack(j0 + 2, 0, CHUNK)
            scatter(1).start(add=True)
            gather(0).start()
            scatter(1).wait()

        gather(0).wait()
        unpack(NFULL - 1, 1, CHUNK)
        scatter(0).start(add=True)
        gather(1).start()
        gather(1).wait()
        scatter(0).wait()
        scatter(1).start(add=True)
        unpack(NFULL, 0, TAIL)
        pltpu.sync_copy(h_hbm.at[src_v.at[0].at[pl.ds(0, TAIL)]],
                        rows_v.at[0].at[pl.ds(0, TAIL)])
        scatter(1).wait()
        pltpu.sync_copy(rows_v.at[0].at[pl.ds(0, TAIL)],
                        agg_sh.at[dst_v.at[0].at[pl.ds(0, TAIL)]], add=True)

        plsc.subcore_barrier()

        @pl.loop(0, 11)
        def _(t):
            b = s + NS * t

            @pl.when(b < NBLK)
            def _():
                pltpu.sync_copy(agg_sh.at[pl.ds(b * RBLK, RBLK)],
                                out_hbm.at[c].at[pl.ds(b * RBLK, RBLK)])

    return k(h, epk)


MLP_BLK = 1352


def _mlp(p, h, W1, b1, W2, b2):
    def body(p_ref, h_ref, w1_ref, b1_ref, w2_ref, b2_ref, o_ref):
        z = p_ref[0] + p_ref[1] + h_ref[...]
        z = jnp.dot(z, w1_ref[...], precision=_HI,
                    preferred_element_type=jnp.float32) + b1_ref[...]
        z = jnp.maximum(z, 0.0)
        z = jnp.dot(z, w2_ref[...], precision=_HI,
                    preferred_element_type=jnp.float32) + b2_ref[...]
        o_ref[...] = jnp.maximum(z, 0.0)

    return pl.pallas_call(
        body,
        grid=(N // MLP_BLK,),
        in_specs=[
            pl.BlockSpec((NC, MLP_BLK, D), lambda i: (0, i, 0)),
            pl.BlockSpec((MLP_BLK, D), lambda i: (i, 0)),
            pl.BlockSpec((D, D), lambda i: (0, 0)),
            pl.BlockSpec((1, D), lambda i: (0, 0)),
            pl.BlockSpec((D, D), lambda i: (0, 0)),
            pl.BlockSpec((1, D), lambda i: (0, 0)),
        ],
        out_specs=pl.BlockSpec((MLP_BLK, D), lambda i: (i, 0)),
        out_shape=jax.ShapeDtypeStruct((N, D), jnp.float32),
    )(p, h, W1, b1, W2, b2)


def _head(h, Wl, bl, Wm, bm, Ws, bs):
    def body(h_ref, wl_ref, bl_ref, wm_ref, bm_ref, ws_ref, bs_ref,
             mean_ref, std_ref):
        node = lax.broadcasted_iota(jnp.int32, (G, N), 1)
        gid = lax.broadcasted_iota(jnp.int32, (G, N), 0)
        P = jnp.where(node // NPG == gid, 1.0, 0.0)
        pooled = jnp.dot(P, h_ref[...], precision=_HI,
                         preferred_element_type=jnp.float32) * (1.0 / NPG)
        feat = jnp.dot(pooled, wl_ref[...], precision=_HI,
                       preferred_element_type=jnp.float32) + bl_ref[...]
        feat = jnp.maximum(feat, 0.0)
        mean_ref[...] = jnp.dot(feat, wm_ref[...], precision=_HI,
                                preferred_element_type=jnp.float32) + bm_ref[...]
        sv = jnp.dot(feat, ws_ref[...], precision=_HI,
                     preferred_element_type=jnp.float32) + bs_ref[...]
        std_ref[...] = jax.nn.softplus(sv)

    return pl.pallas_call(
        body,
        out_shape=(jax.ShapeDtypeStruct((G, 32), jnp.float32),
                   jax.ShapeDtypeStruct((G, 32), jnp.float32)),
    )(h, Wl, bl, Wm, bm, Ws, bs)


def kernel(x, edge_index, W1, b1, W2, b2, Wl, bl, Wm, bm, Ws, bs):
    src = edge_index[0].reshape(NW, EPW)
    dst = edge_index[1].reshape(NW, EPW)
    packed = jnp.bitwise_or(src, jnp.left_shift(dst, 16))
    pad = jnp.zeros((NW, NPAD), jnp.int32)
    epk = jnp.concatenate([packed, pad], axis=1).reshape(NW, NCH, CHUNK)
    h = x
    for l in range(3):
        p = _sc_aggregate(h, epk)
        h = _mlp(p, h, W1[l], b1[l].reshape(1, D), W2[l], b2[l].reshape(1, D))
    mean, std = _head(h, Wl, bl.reshape(1, -1), Wm, bm.reshape(1, -1),
                      Ws, bs.reshape(1, -1))
    return (mean, std)


# speedup vs baseline: 2.5644x; 1.0446x over previous
import functools

import jax
import jax.numpy as jnp
from jax import lax
from jax.experimental import pallas as pl
from jax.experimental.pallas import tpu as pltpu
from jax.experimental.pallas import tpu_sc as plsc

N = 10816
E = 346112
D = 128
G = 64
NPG = 169
NC = 2
NS = 16
NW = NC * NS
EPW = E // NW
CHUNK = 64
NCH = EPW // CHUNK
NEPK = (NCH + 1) // 2
NSLOT = 3
NACC = N
RBLK = 64
NBLK = N // RBLK

_HI = lax.Precision.HIGHEST


def _sc_aggregate(h, epk):
    mesh = plsc.VectorSubcoreMesh(core_axis_name="c", subcore_axis_name="s")

    @functools.partial(
        pl.kernel,
        out_type=jax.ShapeDtypeStruct((NC, N, D), jnp.float32),
        mesh=mesh,
        scratch_types=[
            pltpu.VMEM((NEPK, 2 * CHUNK), jnp.int32),
            pltpu.VMEM((NSLOT, CHUNK), jnp.int32),
            pltpu.VMEM((NSLOT, CHUNK), jnp.int32),
            pltpu.VMEM((NSLOT, CHUNK, D), jnp.float32),
            pltpu.VMEM_SHARED((NACC, D), jnp.float32),
            pltpu.SemaphoreType.DMA,
            pltpu.SemaphoreType.DMA,
            pltpu.SemaphoreType.DMA,
            pltpu.SemaphoreType.DMA,
            pltpu.SemaphoreType.DMA,
            pltpu.SemaphoreType.DMA,
        ],
    )
    def k(h_hbm, epk_hbm, out_hbm, epk_v, src_v, dst_v, rows_v, agg_sh,
          gsem0, gsem1, gsem2, ssem0, ssem1, ssem2):
        c = lax.axis_index("c")
        s = lax.axis_index("s")
        wid = c * NS + s
        gsem = (gsem0, gsem1, gsem2)
        ssem = (ssem0, ssem1, ssem2)

        pltpu.sync_copy(epk_hbm.at[wid], epk_v)

        mask16 = jnp.full((16,), 0xFFFF, jnp.int32)

        def gather(b):
            return pltpu.make_async_copy(
                h_hbm.at[src_v.at[b]], rows_v.at[b], gsem[b])

        def scatter(b):
            return pltpu.make_async_copy(
                rows_v.at[b], agg_sh.at[dst_v.at[b]], ssem[b])

        @pl.loop(0, CHUNK)
        def _(r):
            for cc in range(0, D, 16):
                rows_v[0, r, pl.ds(cc, 16)] = jnp.zeros((16,), jnp.float32)

        @pl.loop(0, 11)
        def _(t):
            b = s + NS * t

            @pl.when(b < NBLK)
            def _():
                pltpu.sync_copy(rows_v.at[0].at[pl.ds(0, RBLK)],
                                agg_sh.at[pl.ds(b * RBLK, RBLK)])

        plsc.subcore_barrier()

        def unpack(j, b):
            row = lax.shift_right_logical(j, 1)
            base = jnp.bitwise_and(j, 1) * CHUNK
            for cc in range(0, CHUNK, 16):
                v = epk_v[row, pl.ds(base + cc, 16)]
                src_v[b, pl.ds(cc, 16)] = jnp.bitwise_and(v, mask16)
                dst_v[b, pl.ds(cc, 16)] = lax.shift_right_logical(v, 16)

        for b in range(NSLOT):
            unpack(jnp.int32(b), b)
            gather(b).start()

        @pl.loop(0, NCH // NSLOT - 1)
        def _(t):
            j0 = NSLOT * t
            for b in range(NSLOT):
                gather(b).wait()
                scatter(b).start(add=True)
            for b in range(NSLOT):
                scatter(b).wait()
                unpack(j0 + NSLOT + b, b)
                gather(b).start()

        for b in range(NSLOT):
            gather(b).wait()
            scatter(b).start(add=True)
        scatter(0).wait()
        unpack(jnp.int32(NCH - 1), 0)
        gather(0).start()
        gather(0).wait()
        scatter(0).start(add=True)
        for b in range(NSLOT):
            scatter(b).wait()

        plsc.subcore_barrier()

        @pl.loop(0, 11)
        def _(t):
            b = s + NS * t

            @pl.when(b < NBLK)
            def _():
                pltpu.sync_copy(agg_sh.at[pl.ds(b * RBLK, RBLK)],
                                out_hbm.at[c].at[pl.ds(b * RBLK, RBLK)])

    return k(h, epk)


MLP_BLK = 1352


def _mlp(p, h, W1, b1, W2, b2):
    def body(p_ref, h_ref, w1_ref, b1_ref, w2_ref, b2_ref, o_ref):
        z = p_ref[0] + p_ref[1] + h_ref[...]
        z = jnp.dot(z, w1_ref[...], precision=_HI,
                    preferred_element_type=jnp.float32) + b1_ref[...]
        z = jnp.maximum(z, 0.0)
        z = jnp.dot(z, w2_ref[...], precision=_HI,
                    preferred_element_type=jnp.float32) + b2_ref[...]
        o_ref[...] = jnp.maximum(z, 0.0)

    return pl.pallas_call(
        body,
        grid=(N // MLP_BLK,),
        in_specs=[
            pl.BlockSpec((NC, MLP_BLK, D), lambda i: (0, i, 0)),
            pl.BlockSpec((MLP_BLK, D), lambda i: (i, 0)),
            pl.BlockSpec((D, D), lambda i: (0, 0)),
            pl.BlockSpec((1, D), lambda i: (0, 0)),
            pl.BlockSpec((D, D), lambda i: (0, 0)),
            pl.BlockSpec((1, D), lambda i: (0, 0)),
        ],
        out_specs=pl.BlockSpec((MLP_BLK, D), lambda i: (i, 0)),
        out_shape=jax.ShapeDtypeStruct((N, D), jnp.float32),
    )(p, h, W1, b1, W2, b2)


def _head(h, Wl, bl, Wm, bm, Ws, bs):
    def body(h_ref, wl_ref, bl_ref, wm_ref, bm_ref, ws_ref, bs_ref,
             mean_ref, std_ref):
        node = lax.broadcasted_iota(jnp.int32, (G, N), 1)
        gid = lax.broadcasted_iota(jnp.int32, (G, N), 0)
        P = jnp.where(node // NPG == gid, 1.0, 0.0)
        pooled = jnp.dot(P, h_ref[...], precision=_HI,
                         preferred_element_type=jnp.float32) * (1.0 / NPG)
        feat = jnp.dot(pooled, wl_ref[...], precision=_HI,
                       preferred_element_type=jnp.float32) + bl_ref[...]
        feat = jnp.maximum(feat, 0.0)
        mean_ref[...] = jnp.dot(feat, wm_ref[...], precision=_HI,
                                preferred_element_type=jnp.float32) + bm_ref[...]
        sv = jnp.dot(feat, ws_ref[...], precision=_HI,
                     preferred_element_type=jnp.float32) + bs_ref[...]
        std_ref[...] = jax.nn.softplus(sv)

    return pl.pallas_call(
        body,
        out_shape=(jax.ShapeDtypeStruct((G, 32), jnp.float32),
                   jax.ShapeDtypeStruct((G, 32), jnp.float32)),
    )(h, Wl, bl, Wm, bm, Ws, bs)


def kernel(x, edge_index, W1, b1, W2, b2, Wl, bl, Wm, bm, Ws, bs):
    src = edge_index[0].reshape(NW, EPW)
    dst = edge_index[1].reshape(NW, EPW)
    packed = jnp.bitwise_or(src, jnp.left_shift(dst, 16))
    pad = jnp.zeros((NW, NEPK * 2 * CHUNK - EPW), jnp.int32)
    epk = jnp.concatenate([packed, pad], axis=1).reshape(NW, NEPK, 2 * CHUNK)
    h = x
    for l in range(3):
        p = _sc_aggregate(h, epk)
        h = _mlp(p, h, W1[l], b1[l].reshape(1, D), W2[l], b2[l].reshape(1, D))
    mean, std = _head(h, Wl, bl.reshape(1, -1), Wm, bm.reshape(1, -1),
                      Ws, bs.reshape(1, -1))
    return (mean, std)
